# baseline (speedup 1.0000x reference)
import functools

import jax
import jax.numpy as jnp
from jax import lax
from jax.experimental import pallas as pl
from jax.experimental.pallas import tpu as pltpu
from jax.experimental.pallas import tpu_sc as plsc

N = 10000
E = 160000
IN_DIM = 256
HID = 256
HEADS = 4
OUT_DIM = 16
NUM_GRAPHS = 16
NEG_SLOPE = 0.2

N_PAD = 10240
ROW_BLK = 512
N_ROW_BLKS = N_PAD // ROW_BLK
F = HEADS * HID
N_SLICES = 8
AC = 16

NC = 2
NS = 16
EPT32 = E // (NC * NS)
EPT16 = E // NS
CH2 = 1000
CH3 = 80
RPT = N_PAD // NS

_MESH = dict(core_axis_name="c", subcore_axis_name="s")


def _proj_body(apply_act, nouts, x_ref, w_ref, s_ref, b_ref, *out_refs):
  xv = x_ref[...]
  if apply_act:
    xv = jnp.maximum(xv + b_ref[...], 0.0)
  o = jnp.dot(xv, w_ref[...], preferred_element_type=jnp.float32)
  for s in range(nouts):
    out_refs[s][...] = o[:, 128 * s:128 * (s + 1)].astype(jnp.bfloat16)
  a = jnp.dot(o, s_ref[...], preferred_element_type=jnp.float32)
  out_refs[nouts][...] = a[:, 0:AC]
  out_refs[nouts + 1][...] = a[:, AC:2 * AC]


def _projection(x, w, s_mat, b_prev, apply_act):
  k = x.shape[1]
  grid = (N_ROW_BLKS,)
  out_shape = ([jax.ShapeDtypeStruct((N_PAD, 128), jnp.bfloat16)
                for _ in range(N_SLICES)]
               + [jax.ShapeDtypeStruct((N_PAD, AC), jnp.float32)] * 2)
  out_specs = ([pl.BlockSpec((ROW_BLK, 128), lambda i: (i, 0))
                for _ in range(N_SLICES)]
               + [pl.BlockSpec((ROW_BLK, AC), lambda i: (i, 0))] * 2)
  return pl.pallas_call(
      functools.partial(_proj_body, apply_act, N_SLICES),
      grid=grid,
      in_specs=[
          pl.BlockSpec((ROW_BLK, k), lambda i: (i, 0)),
          pl.BlockSpec((k, F), lambda i: (0, 0)),
          pl.BlockSpec((F, 2 * AC), lambda i: (0, 0)),
          pl.BlockSpec((1, k), lambda i: (0, 0)),
      ],
      out_specs=out_specs,
      out_shape=out_shape,
  )(x, w, s_mat, b_prev)


def _edge_stats_body(src_h, dst_h, as_h, ad_h, ex_h, den0_h, den1_h,
                     srcv, dstv, asg, adg, exb, zb, den_acc, sem1, sem2):
  cid = lax.axis_index("c")
  sid = lax.axis_index("s")
  base = (cid * NS + sid) * EPT32

  def _zrow(i, _):
    zb[i, :] = jnp.zeros((AC,), jnp.float32)
    return 0
  lax.fori_loop(0, RPT, _zrow, 0)
  pltpu.sync_copy(zb, den_acc.at[pl.ds(sid * RPT, RPT)])
  plsc.subcore_barrier()

  for c in range(EPT32 // CH2):
    b = base + c * CH2
    pltpu.sync_copy(src_h.at[pl.ds(b, CH2)], srcv)
    pltpu.sync_copy(dst_h.at[pl.ds(b, CH2)], dstv)
    cp1 = pltpu.async_copy(as_h.at[srcv], asg, sem1)
    cp2 = pltpu.async_copy(ad_h.at[dstv], adg, sem2)
    cp1.wait()
    cp2.wait()

    def _estep(e, _):
      v = asg[e, :] + adg[e, :]
      v = jnp.where(v > 0, v, NEG_SLOPE * v)
      exb[e, :] = jnp.exp(v)
      return 0
    lax.fori_loop(0, CH2, _estep, 0)

    pltpu.sync_copy(exb, ex_h.at[pl.ds(b, CH2)])
    pltpu.sync_copy(exb, den_acc.at[dstv], add=True)

  plsc.subcore_barrier()
  r0 = sid * RPT

  @pl.when(cid == 0)
  def _():
    pltpu.sync_copy(den_acc.at[pl.ds(r0, RPT)],
                    den0_h.at[pl.ds(r0, RPT)])

  @pl.when(cid == 1)
  def _():
    pltpu.sync_copy(den_acc.at[pl.ds(r0, RPT)],
                    den1_h.at[pl.ds(r0, RPT)])


_edge_stats = pl.kernel(
    _edge_stats_body,
    out_type=[jax.ShapeDtypeStruct((E, AC), jnp.float32),
              jax.ShapeDtypeStruct((N_PAD, AC), jnp.float32),
              jax.ShapeDtypeStruct((N_PAD, AC), jnp.float32)],
    mesh=plsc.VectorSubcoreMesh(**_MESH),
    compiler_params=pltpu.CompilerParams(use_tc_tiling_on_sc=False, needs_layout_passes=False),
    scratch_types=[
        pltpu.VMEM((CH2,), jnp.int32),
        pltpu.VMEM((CH2,), jnp.int32),
        pltpu.VMEM((CH2, AC), jnp.float32),
        pltpu.VMEM((CH2, AC), jnp.float32),
        pltpu.VMEM((CH2, AC), jnp.float32),
        pltpu.VMEM((RPT, AC), jnp.float32),
        pltpu.VMEM_SHARED((N_PAD, AC), jnp.float32),
        pltpu.SemaphoreType.DMA,
        pltpu.SemaphoreType.DMA,
    ],
)


NTG = -(-CH2 // 16)
CH2P = NTG * 16


def _alpha_body(ex_h, den0_h, den1_h, dst_h, alt_h,
                dstv, exv, d0g, d1g, alb, albt, sem1, sem2):
  cid = lax.axis_index("c")
  sid = lax.axis_index("s")
  base = (cid * NS + sid) * EPT32

  for c in range(EPT32 // CH2):
    b = base + c * CH2
    pltpu.sync_copy(dst_h.at[pl.ds(b, CH2)], dstv)
    pltpu.sync_copy(ex_h.at[pl.ds(b, CH2)], exv)
    cp1 = pltpu.async_copy(den0_h.at[dstv], d0g, sem1)
    cp2 = pltpu.async_copy(den1_h.at[dstv], d1g, sem2)
    cp1.wait()
    cp2.wait()

    def _astep(e, _):
      den = d0g[e, :] + d1g[e, :] + 1e-16
      alb[e, :] = exv[e, :] / den
      return 0
    lax.fori_loop(0, CH2, _astep, 0)

    for h in range(HEADS):
      def _tg(g, _):
        rowi = jnp.minimum(16 * g + lax.iota(jnp.int32, 16), CH2 - 1)
        v = plsc.load_gather(alb, [rowi, jnp.full((16,), h, jnp.int32)])
        albt[h, pl.ds(16 * g, 16)] = v
        return 0
      lax.fori_loop(0, NTG, _tg, 0)
      pltpu.sync_copy(albt.at[h, pl.ds(0, CH2)], alt_h.at[h, pl.ds(b, CH2)])


_alpha_kernel = pl.kernel(
    _alpha_body,
    out_type=jax.ShapeDtypeStruct((HEADS, E), jnp.float32),
    mesh=plsc.VectorSubcoreMesh(**_MESH),
    compiler_params=pltpu.CompilerParams(use_tc_tiling_on_sc=False, needs_layout_passes=False),
    scratch_types=[
        pltpu.VMEM((CH2,), jnp.int32),
        pltpu.VMEM((CH2, AC), jnp.float32),
        pltpu.VMEM((CH2, AC), jnp.float32),
        pltpu.VMEM((CH2, AC), jnp.float32),
        pltpu.VMEM((CH2, AC), jnp.float32),
        pltpu.VMEM((HEADS, CH2P), jnp.float32),
        pltpu.SemaphoreType.DMA,
        pltpu.SemaphoreType.DMA,
    ],
)


SUP = 2000
NSUP = EPT16 // SUP
NCH = SUP // CH3
NB = 3
NSR = 2
ROUND = NB * NSR
NQ = (NCH - 1) // ROUND


def _slice_pass(s, h_refs, o_refs, src_h, dst_h, alt_h, base, sid, acc,
                srcs, dsts, alb, asp, dstvs, rowss, srows, gsems, ssems):
  head = s // 2
  hs = h_refs[s]

  def _prep(c, dstv):
    def _grp(g, _):
      off = CH3 * c + 16 * g
      a16 = alb[pl.ds(off, 16)]
      for l in range(16):
        asp[16 * g + l, :] = jnp.full((AC,), a16[l], jnp.float32)
      dstv[pl.ds(16 * g, 16)] = dsts[pl.ds(off, 16)]
      return 0
    lax.fori_loop(0, CH3 // 16, _grp, 0)

  def _scale(rows_bf, sr):
    def _e(e, _):
      av = asp[e, :]
      for g4 in range(4):
        w = rows_bf[e, pl.ds(32 * g4, 32)]
        ev, od = plsc.unpack(w, format=plsc.PackFormat.INTERLEAVED)
        sr[e, pl.ds(32 * g4, 16)] = ev * av
        sr[e, pl.ds(32 * g4 + 16, 16)] = od * av
      return 0
    lax.fori_loop(0, CH3, _e, 0)

  def _gissue(c, b):
    idx = srcs.at[pl.ds(CH3 * c, CH3)]
    pltpu.async_copy(hs.at[idx], rowss[b], gsems[b])

  def _gwait(c, b):
    idx = srcs.at[pl.ds(CH3 * c, CH3)]
    pltpu.make_async_copy(hs.at[idx], rowss[b], gsems[b]).wait()

  def _sissue(b):
    pltpu.async_copy(srows[b], acc.at[dstvs[b]], ssems[b], add=True)

  def _swait(b):
    pltpu.make_async_copy(srows[b], acc.at[dstvs[b]], ssems[b]).wait()

  def _super(sc, _):
    eb = base + SUP * sc
    pltpu.sync_copy(alt_h.at[head, pl.ds(eb, SUP)], alb)
    pltpu.sync_copy(src_h.at[pl.ds(eb, SUP)], srcs)
    pltpu.sync_copy(dst_h.at[pl.ds(eb, SUP)], dsts)
    _gissue(0, 0)
    _gissue(1, 1)

    def _round(q, _):
      for p in range(ROUND):
        c = ROUND * q + p
        p3 = p % NB
        p2 = p % NSR

        if p >= 2:
          _swait(p2)
        else:
          @pl.when(q >= 1)
          def _():
            _swait(p2)

        _prep(c, dstvs[p2])
        _gwait(c, p3)
        _scale(rowss[p3], srows[p2])

        if p == ROUND - 1:
          @pl.when(q <= NQ - 2)
          def _():
            _gissue(c + 2, (p3 + 2) % NB)
        else:
          _gissue(c + 2, (p3 + 2) % NB)

        _sissue(p2)
      return 0

    lax.fori_loop(0, NQ, _round, 0)

    _swait(0)
    _prep(NCH - 1, dstvs[0])
    _gwait(NCH - 1, 0)
    _scale(rowss[0], srows[0])
    _sissue(0)
    _swait(1)
    _swait(0)
    return 0

  lax.fori_loop(0, NSUP, _super, 0)
  plsc.subcore_barrier()
  r0 = sid * RPT
  pltpu.sync_copy(acc.at[pl.ds(r0, RPT)], o_refs[s].at[pl.ds(r0, RPT)])


def _aggregate_body(*args):
  h_refs = args[0:N_SLICES]
  src_h, dst_h, alt_h = args[N_SLICES:N_SLICES + 3]
  o_refs = args[N_SLICES + 3:2 * N_SLICES + 3]
  rest = args[2 * N_SLICES + 3:]
  srcs, dsts, alb, asp = rest[0:4]
  dstvs = rest[4:4 + NSR]
  rowss = rest[4 + NSR:4 + NSR + NB]
  srows = rest[4 + NSR + NB:4 + 2 * NSR + NB]
  acc = rest[4 + 2 * NSR + NB]
  gsems = rest[5 + 2 * NSR + NB:5 + 2 * NSR + 2 * NB]
  ssems = rest[5 + 2 * NSR + 2 * NB:5 + 3 * NSR + 2 * NB]

  cid = lax.axis_index("c")
  sid = lax.axis_index("s")
  base = sid * EPT16

  for s_loc in range(N_SLICES // NC):
    def _zrow(i, _):
      for cbl in range(8):
        srows[0][i, pl.ds(16 * cbl, 16)] = jnp.zeros((16,), jnp.float32)
      return 0
    lax.fori_loop(0, CH3, _zrow, 0)
    for k in range(RPT // CH3):
      pltpu.sync_copy(srows[0], acc.at[pl.ds(sid * RPT + k * CH3, CH3)])
    plsc.subcore_barrier()

    @pl.when(cid == 0)
    def _():
      _slice_pass(s_loc, h_refs, o_refs, src_h, dst_h, alt_h, base, sid,
                  acc, srcs, dsts, alb, asp, dstvs, rowss, srows,
                  gsems, ssems)

    @pl.when(cid == 1)
    def _():
      _slice_pass(4 + s_loc, h_refs, o_refs, src_h, dst_h, alt_h, base,
                  sid, acc, srcs, dsts, alb, asp, dstvs, rowss, srows,
                  gsems, ssems)

    plsc.subcore_barrier()


_aggregate = pl.kernel(
    _aggregate_body,
    out_type=[jax.ShapeDtypeStruct((N_PAD, 128), jnp.float32)
              for _ in range(N_SLICES)],
    mesh=plsc.VectorSubcoreMesh(**_MESH),
    compiler_params=pltpu.CompilerParams(use_tc_tiling_on_sc=False, needs_layout_passes=False),
    scratch_types=(
        [pltpu.VMEM((SUP,), jnp.int32),
         pltpu.VMEM((SUP,), jnp.int32),
         pltpu.VMEM((SUP,), jnp.float32),
         pltpu.VMEM((CH3, AC), jnp.float32)]
        + [pltpu.VMEM((CH3,), jnp.int32) for _ in range(NSR)]
        + [pltpu.VMEM((CH3, 128), jnp.bfloat16) for _ in range(NB)]
        + [pltpu.VMEM((CH3, 128), jnp.float32) for _ in range(NSR)]
        + [pltpu.VMEM_SHARED((N_PAD, 128), jnp.float32)]
        + [pltpu.SemaphoreType.DMA for _ in range(NB)]
        + [pltpu.SemaphoreType.DMA for _ in range(NSR)]
    ),
)


def _pool_body(h_refs, b2_ref, batch_ref, wc1_ref, bc1_ref, wc2_ref, bc2_ref,
               wc3_ref, bc3_ref, out_ref, s_acc, m_acc, c_acc, z_scr):
  i = pl.program_id(0)

  @pl.when(i == 0)
  def _():
    s_acc[...] = jnp.zeros_like(s_acc)
    m_acc[...] = jnp.full_like(m_acc, -1e30)
    c_acc[...] = jnp.zeros_like(c_acc)

  bt = batch_ref[...]
  b2 = b2_ref[...]
  for g in range(NUM_GRAPHS):
    mk = bt == g
    c_acc[g:g + 1, :] = c_acc[g:g + 1, :] + jnp.sum(mk.astype(jnp.float32))
    for s in range(N_SLICES):
      hs = jnp.maximum(h_refs[s][...] + b2[:, 128 * s:128 * (s + 1)], 0.0)
      sp = jnp.sum(jnp.where(mk, hs, 0.0), axis=0, keepdims=True)
      s_acc[g:g + 1, 128 * s:128 * (s + 1)] = (
          s_acc[g:g + 1, 128 * s:128 * (s + 1)] + sp)
      mp = jnp.max(jnp.where(mk, hs, -1e30), axis=0, keepdims=True)
      m_acc[g:g + 1, 128 * s:128 * (s + 1)] = jnp.maximum(
          m_acc[g:g + 1, 128 * s:128 * (s + 1)], mp)

  @pl.when(i == N_ROW_BLKS - 1)
  def _():
    cnt = jnp.maximum(c_acc[...], 1.0)
    sv = s_acc[...]
    mfix = jnp.where(m_acc[...] > -5e29, m_acc[...], 0.0)
    z_scr[:, 0:F] = sv / cnt[:, 0:1]
    z_scr[:, F:2 * F] = mfix
    z_scr[:, 2 * F:3 * F] = sv
    z1 = jnp.maximum(
        jnp.dot(z_scr[...], wc1_ref[...], preferred_element_type=jnp.float32)
        + bc1_ref[...], 0.0)
    z2 = jnp.maximum(
        jnp.dot(z1, wc2_ref[...], preferred_element_type=jnp.float32)
        + bc2_ref[...], 0.0)
    out_ref[...] = (jnp.dot(z2, wc3_ref[...],
                            preferred_element_type=jnp.float32)
                    + bc3_ref[...])


def _pool_mlp(h_list, b2, batch2, wc1, bc1, wc2, bc2, wc3, bc3):
  def body(*refs):
    _pool_body(refs[0:N_SLICES], *refs[N_SLICES:])

  hid2 = wc1.shape[1]
  in_specs = ([pl.BlockSpec((ROW_BLK, 128), lambda i: (i, 0))
               for _ in range(N_SLICES)] + [
      pl.BlockSpec((1, F), lambda i: (0, 0)),
      pl.BlockSpec((ROW_BLK, 1), lambda i: (i, 0)),
      pl.BlockSpec(wc1.shape, lambda i: (0, 0)),
      pl.BlockSpec((1, hid2), lambda i: (0, 0)),
      pl.BlockSpec(wc2.shape, lambda i: (0, 0)),
      pl.BlockSpec((1, HID), lambda i: (0, 0)),
      pl.BlockSpec(wc3.shape, lambda i: (0, 0)),
      pl.BlockSpec((1, OUT_DIM), lambda i: (0, 0)),
  ])
  return pl.pallas_call(
      body,
      grid=(N_ROW_BLKS,),
      in_specs=in_specs,
      out_specs=pl.BlockSpec((NUM_GRAPHS, OUT_DIM), lambda i: (0, 0)),
      out_shape=jax.ShapeDtypeStruct((NUM_GRAPHS, OUT_DIM), jnp.float32),
      scratch_shapes=[
          pltpu.VMEM((NUM_GRAPHS, F), jnp.float32),
          pltpu.VMEM((NUM_GRAPHS, F), jnp.float32),
          pltpu.VMEM((NUM_GRAPHS, 128), jnp.float32),
          pltpu.VMEM((NUM_GRAPHS, 3 * F), jnp.float32),
      ],
  )(*h_list, b2.reshape(1, F), batch2, wc1, bc1.reshape(1, hid2),
    wc2, bc2.reshape(1, HID), wc3, bc3.reshape(1, OUT_DIM))


def _att_matrix(att):
  rows = jnp.arange(F) // HID
  cols = jnp.arange(AC) % HEADS
  att_flat = att.reshape(F)
  return jnp.where(rows[:, None] == cols[None, :], att_flat[:, None], 0.0)


def _sigma():
  j = jnp.arange(F)
  grp = (j // 32) * 32
  pos = j % 32
  return grp + jnp.where(pos % 2 == 0, pos // 2, 16 + pos // 2)


def _gat_layer(x, src, dst, w, a_s, a_d, b_prev, apply_act):
  sig = _sigma()
  s_mat = jnp.concatenate([_att_matrix(a_s), _att_matrix(a_d)], axis=1)[sig]
  outs = _projection(x, w[:, sig], s_mat, b_prev.reshape(1, -1), apply_act)
  h_list, as_t, ad_t = outs[:N_SLICES], outs[N_SLICES], outs[N_SLICES + 1]
  ex, den0, den1 = _edge_stats(src, dst, as_t, ad_t)
  alpha = _alpha_kernel(ex, den0, den1, dst)
  return _aggregate(*h_list, src, dst, alpha)


def kernel(x, edge_index, batch, W0, a_s0, a_d0, b0, W1, a_s1, a_d1, b1,
           W2, a_s2, a_d2, b2, Wc1, bc1, Wc2, bc2, Wc3, bc3):
  src, dst = edge_index[0], edge_index[1]
  x_pad = jnp.pad(x, ((0, N_PAD - N), (0, 0)))
  batch2 = jnp.pad(batch, (0, N_PAD - N),
                   constant_values=NUM_GRAPHS).reshape(N_PAD, 1)

  o1 = _gat_layer(x_pad, src, dst, W0, a_s0, a_d0,
                  jnp.zeros((IN_DIM,), jnp.float32), False)
  x1 = jnp.concatenate(o1, axis=1)
  o2 = _gat_layer(x1, src, dst, W1, a_s1, a_d1, b0, True)
  x2 = jnp.concatenate(o2, axis=1)
  o3 = _gat_layer(x2, src, dst, W2, a_s2, a_d2, b1, True)

  return _pool_mlp(o3, b2, batch2, Wc1, bc1, Wc2, bc2, Wc3, bc3)


# speedup vs baseline: 1.5581x; 1.5581x over previous
import functools

import jax
import jax.numpy as jnp
from jax import lax
from jax.experimental import pallas as pl
from jax.experimental.pallas import tpu as pltpu
from jax.experimental.pallas import tpu_sc as plsc

N = 10000
E = 160000
IN_DIM = 256
HID = 256
HEADS = 4
OUT_DIM = 16
NUM_GRAPHS = 16
NEG_SLOPE = 0.2

N_PAD = 10240
ROW_BLK = 512
N_ROW_BLKS = N_PAD // ROW_BLK
F = HEADS * HID
N_SLICES = 8
AC = 16

NC = 2
NS = 16
EPT32 = E // (NC * NS)
EPT16 = E // NS
CH2 = 1000
CH3 = 80
RPT = N_PAD // NS

_MESH = dict(core_axis_name="c", subcore_axis_name="s")


def _proj_body(apply_act, nouts, x_ref, w_ref, s_ref, b_ref, *out_refs):
  xv = x_ref[...]
  if apply_act:
    xv = jnp.maximum(xv + b_ref[...], 0.0)
  o = jnp.dot(xv, w_ref[...], preferred_element_type=jnp.float32)
  for s in range(nouts):
    out_refs[s][...] = o[:, 128 * s:128 * (s + 1)]
  a = jnp.dot(o, s_ref[...], preferred_element_type=jnp.float32)
  out_refs[nouts][...] = a[:, 0:AC]
  out_refs[nouts + 1][...] = a[:, AC:2 * AC]


def _projection(x, w, s_mat, b_prev, apply_act):
  k = x.shape[1]
  grid = (N_ROW_BLKS,)
  out_shape = ([jax.ShapeDtypeStruct((N_PAD, 128), jnp.float32)
                for _ in range(N_SLICES)]
               + [jax.ShapeDtypeStruct((N_PAD, AC), jnp.float32)] * 2)
  out_specs = ([pl.BlockSpec((ROW_BLK, 128), lambda i: (i, 0))
                for _ in range(N_SLICES)]
               + [pl.BlockSpec((ROW_BLK, AC), lambda i: (i, 0))] * 2)
  return pl.pallas_call(
      functools.partial(_proj_body, apply_act, N_SLICES),
      grid=grid,
      in_specs=[
          pl.BlockSpec((ROW_BLK, k), lambda i: (i, 0)),
          pl.BlockSpec((k, F), lambda i: (0, 0)),
          pl.BlockSpec((F, 2 * AC), lambda i: (0, 0)),
          pl.BlockSpec((1, k), lambda i: (0, 0)),
      ],
      out_specs=out_specs,
      out_shape=out_shape,
  )(x, w, s_mat, b_prev)


def _edge_stats_body(src_h, dst_h, as_h, ad_h, ex_h, den0_h, den1_h,
                     srcv, dstv, asg, adg, exb, zb, den_acc, sem1, sem2):
  cid = lax.axis_index("c")
  sid = lax.axis_index("s")
  base = (cid * NS + sid) * EPT32

  def _zrow(i, _):
    zb[i, :] = jnp.zeros((AC,), jnp.float32)
    return 0
  lax.fori_loop(0, RPT, _zrow, 0)
  pltpu.sync_copy(zb, den_acc.at[pl.ds(sid * RPT, RPT)])
  plsc.subcore_barrier()

  for c in range(EPT32 // CH2):
    b = base + c * CH2
    pltpu.sync_copy(src_h.at[pl.ds(b, CH2)], srcv)
    pltpu.sync_copy(dst_h.at[pl.ds(b, CH2)], dstv)
    cp1 = pltpu.async_copy(as_h.at[srcv], asg, sem1)
    cp2 = pltpu.async_copy(ad_h.at[dstv], adg, sem2)
    cp1.wait()
    cp2.wait()

    def _estep(e, _):
      v = asg[e, :] + adg[e, :]
      v = jnp.where(v > 0, v, NEG_SLOPE * v)
      exb[e, :] = jnp.exp(v)
      return 0
    lax.fori_loop(0, CH2, _estep, 0)

    pltpu.sync_copy(exb, ex_h.at[pl.ds(b, CH2)])
    pltpu.sync_copy(exb, den_acc.at[dstv], add=True)

  plsc.subcore_barrier()
  r0 = sid * RPT

  @pl.when(cid == 0)
  def _():
    pltpu.sync_copy(den_acc.at[pl.ds(r0, RPT)],
                    den0_h.at[pl.ds(r0, RPT)])

  @pl.when(cid == 1)
  def _():
    pltpu.sync_copy(den_acc.at[pl.ds(r0, RPT)],
                    den1_h.at[pl.ds(r0, RPT)])


_edge_stats = pl.kernel(
    _edge_stats_body,
    out_type=[jax.ShapeDtypeStruct((E, AC), jnp.float32),
              jax.ShapeDtypeStruct((N_PAD, AC), jnp.float32),
              jax.ShapeDtypeStruct((N_PAD, AC), jnp.float32)],
    mesh=plsc.VectorSubcoreMesh(**_MESH),
    compiler_params=pltpu.CompilerParams(use_tc_tiling_on_sc=False, needs_layout_passes=False),
    scratch_types=[
        pltpu.VMEM((CH2,), jnp.int32),
        pltpu.VMEM((CH2,), jnp.int32),
        pltpu.VMEM((CH2, AC), jnp.float32),
        pltpu.VMEM((CH2, AC), jnp.float32),
        pltpu.VMEM((CH2, AC), jnp.float32),
        pltpu.VMEM((RPT, AC), jnp.float32),
        pltpu.VMEM_SHARED((N_PAD, AC), jnp.float32),
        pltpu.SemaphoreType.DMA,
        pltpu.SemaphoreType.DMA,
    ],
)


NTG = -(-CH2 // 16)
CH2P = NTG * 16


def _alpha_body(ex_h, den0_h, den1_h, dst_h, alt_h,
                dstv, exv, d0g, d1g, alb, albt, sem1, sem2):
  cid = lax.axis_index("c")
  sid = lax.axis_index("s")
  base = (cid * NS + sid) * EPT32

  for c in range(EPT32 // CH2):
    b = base + c * CH2
    pltpu.sync_copy(dst_h.at[pl.ds(b, CH2)], dstv)
    pltpu.sync_copy(ex_h.at[pl.ds(b, CH2)], exv)
    cp1 = pltpu.async_copy(den0_h.at[dstv], d0g, sem1)
    cp2 = pltpu.async_copy(den1_h.at[dstv], d1g, sem2)
    cp1.wait()
    cp2.wait()

    def _astep(e, _):
      den = d0g[e, :] + d1g[e, :] + 1e-16
      alb[e, :] = exv[e, :] / den
      return 0
    lax.fori_loop(0, CH2, _astep, 0)

    for h in range(HEADS):
      def _tg(g, _):
        rowi = jnp.minimum(16 * g + lax.iota(jnp.int32, 16), CH2 - 1)
        v = plsc.load_gather(alb, [rowi, jnp.full((16,), h, jnp.int32)])
        albt[h, pl.ds(16 * g, 16)] = v
        return 0
      lax.fori_loop(0, NTG, _tg, 0)
      pltpu.sync_copy(albt.at[h, pl.ds(0, CH2)], alt_h.at[h, pl.ds(b, CH2)])


_alpha_kernel = pl.kernel(
    _alpha_body,
    out_type=jax.ShapeDtypeStruct((HEADS, E), jnp.float32),
    mesh=plsc.VectorSubcoreMesh(**_MESH),
    compiler_params=pltpu.CompilerParams(use_tc_tiling_on_sc=False, needs_layout_passes=False),
    scratch_types=[
        pltpu.VMEM((CH2,), jnp.int32),
        pltpu.VMEM((CH2, AC), jnp.float32),
        pltpu.VMEM((CH2, AC), jnp.float32),
        pltpu.VMEM((CH2, AC), jnp.float32),
        pltpu.VMEM((CH2, AC), jnp.float32),
        pltpu.VMEM((HEADS, CH2P), jnp.float32),
        pltpu.SemaphoreType.DMA,
        pltpu.SemaphoreType.DMA,
    ],
)


SUP = 2000
NSUP = EPT16 // SUP
NCH = SUP // CH3
NB = 3
NQ = (NCH - 1) // NB


def _slice_pass(s, h_refs, o_refs, src_h, dst_h, alt_h, base, sid, acc,
                srcs, dsts, alb, asp, dstvs, rowss, gsems, ssems):
  head = s // 2
  hs = h_refs[s]

  def _prep(c, dstv):
    def _grp(g, _):
      off = CH3 * c + 16 * g
      a16 = alb[pl.ds(off, 16)]
      for l in range(16):
        asp[16 * g + l, :] = jnp.full((AC,), a16[l], jnp.float32)
      dstv[pl.ds(16 * g, 16)] = dsts[pl.ds(off, 16)]
      return 0
    lax.fori_loop(0, CH3 // 16, _grp, 0)

  def _scale(rows):
    def _e(e, _):
      av = asp[e, :]
      for cbl in range(8):
        rows[e, pl.ds(16 * cbl, 16)] = rows[e, pl.ds(16 * cbl, 16)] * av
      return 0
    lax.fori_loop(0, CH3, _e, 0, unroll=4)

  def _gissue(c, b):
    idx = srcs.at[pl.ds(CH3 * c, CH3)]
    pltpu.async_copy(hs.at[idx], rowss[b], gsems[b])

  def _gwait(c, b):
    idx = srcs.at[pl.ds(CH3 * c, CH3)]
    pltpu.make_async_copy(hs.at[idx], rowss[b], gsems[b]).wait()

  def _sissue(b):
    pltpu.async_copy(rowss[b], acc.at[dstvs[b]], ssems[b], add=True)

  def _swait(b):
    pltpu.make_async_copy(rowss[b], acc.at[dstvs[b]], ssems[b]).wait()

  def _super(sc, _):
    eb = base + SUP * sc
    pltpu.sync_copy(alt_h.at[head, pl.ds(eb, SUP)], alb)
    pltpu.sync_copy(src_h.at[pl.ds(eb, SUP)], srcs)
    pltpu.sync_copy(dst_h.at[pl.ds(eb, SUP)], dsts)
    _gissue(0, 0)
    _gissue(1, 1)

    def _round(q, _):
      for p in range(NB):
        c = NB * q + p
        pa = (p + 2) % NB
        qmax = (NCH - 3 - p) // NB

        _prep(c, dstvs[p])
        _gwait(c, p)
        _scale(rowss[p])

        @pl.when(q <= qmax)
        def _():
          if p == 0:
            @pl.when(q >= 1)
            def _():
              _swait(pa)
          else:
            _swait(pa)
          _gissue(c + 2, pa)

        _sissue(p)
      return 0

    lax.fori_loop(0, NQ, _round, 0)

    _prep(NCH - 1, dstvs[0])
    _gwait(NCH - 1, 0)
    _scale(rowss[0])
    _sissue(0)
    for b in range(NB):
      _swait(b)
    return 0

  lax.fori_loop(0, NSUP, _super, 0)
  plsc.subcore_barrier()
  r0 = sid * RPT
  pltpu.sync_copy(acc.at[pl.ds(r0, RPT)], o_refs[s].at[pl.ds(r0, RPT)])


def _aggregate_body(*args):
  h_refs = args[0:N_SLICES]
  src_h, dst_h, alt_h = args[N_SLICES:N_SLICES + 3]
  o_refs = args[N_SLICES + 3:2 * N_SLICES + 3]
  rest = args[2 * N_SLICES + 3:]
  srcs, dsts, alb, asp = rest[0:4]
  dstvs = rest[4:4 + NB]
  rowss = rest[4 + NB:4 + 2 * NB]
  acc = rest[4 + 2 * NB]
  gsems = rest[5 + 2 * NB:5 + 3 * NB]
  ssems = rest[5 + 3 * NB:5 + 4 * NB]

  cid = lax.axis_index("c")
  sid = lax.axis_index("s")
  base = sid * EPT16

  for s_loc in range(N_SLICES // NC):
    def _zrow(i, _):
      for cbl in range(8):
        rowss[0][i, pl.ds(16 * cbl, 16)] = jnp.zeros((16,), jnp.float32)
      return 0
    lax.fori_loop(0, CH3, _zrow, 0)
    for k in range(RPT // CH3):
      pltpu.sync_copy(rowss[0], acc.at[pl.ds(sid * RPT + k * CH3, CH3)])
    plsc.subcore_barrier()

    @pl.when(cid == 0)
    def _():
      _slice_pass(s_loc, h_refs, o_refs, src_h, dst_h, alt_h, base, sid,
                  acc, srcs, dsts, alb, asp, dstvs, rowss, gsems, ssems)

    @pl.when(cid == 1)
    def _():
      _slice_pass(4 + s_loc, h_refs, o_refs, src_h, dst_h, alt_h, base, sid,
                  acc, srcs, dsts, alb, asp, dstvs, rowss, gsems, ssems)

    plsc.subcore_barrier()


_aggregate = pl.kernel(
    _aggregate_body,
    out_type=[jax.ShapeDtypeStruct((N_PAD, 128), jnp.float32)
              for _ in range(N_SLICES)],
    mesh=plsc.VectorSubcoreMesh(**_MESH),
    compiler_params=pltpu.CompilerParams(use_tc_tiling_on_sc=False, needs_layout_passes=False),
    scratch_types=(
        [pltpu.VMEM((SUP,), jnp.int32),
         pltpu.VMEM((SUP,), jnp.int32),
         pltpu.VMEM((SUP,), jnp.float32),
         pltpu.VMEM((CH3, AC), jnp.float32)]
        + [pltpu.VMEM((CH3,), jnp.int32) for _ in range(NB)]
        + [pltpu.VMEM((CH3, 128), jnp.float32) for _ in range(NB)]
        + [pltpu.VMEM_SHARED((N_PAD, 128), jnp.float32)]
        + [pltpu.SemaphoreType.DMA for _ in range(2 * NB)]
    ),
)


def _pool_body(h_refs, b2_ref, batch_ref, wc1_ref, bc1_ref, wc2_ref, bc2_ref,
               wc3_ref, bc3_ref, out_ref, s_acc, m_acc, c_acc, z_scr):
  i = pl.program_id(0)

  @pl.when(i == 0)
  def _():
    s_acc[...] = jnp.zeros_like(s_acc)
    m_acc[...] = jnp.full_like(m_acc, -1e30)
    c_acc[...] = jnp.zeros_like(c_acc)

  bt = batch_ref[...]
  b2 = b2_ref[...]
  for g in range(NUM_GRAPHS):
    mk = bt == g
    c_acc[g:g + 1, :] = c_acc[g:g + 1, :] + jnp.sum(mk.astype(jnp.float32))
    for s in range(N_SLICES):
      hs = jnp.maximum(h_refs[s][...] + b2[:, 128 * s:128 * (s + 1)], 0.0)
      sp = jnp.sum(jnp.where(mk, hs, 0.0), axis=0, keepdims=True)
      s_acc[g:g + 1, 128 * s:128 * (s + 1)] = (
          s_acc[g:g + 1, 128 * s:128 * (s + 1)] + sp)
      mp = jnp.max(jnp.where(mk, hs, -1e30), axis=0, keepdims=True)
      m_acc[g:g + 1, 128 * s:128 * (s + 1)] = jnp.maximum(
          m_acc[g:g + 1, 128 * s:128 * (s + 1)], mp)

  @pl.when(i == N_ROW_BLKS - 1)
  def _():
    cnt = jnp.maximum(c_acc[...], 1.0)
    sv = s_acc[...]
    mfix = jnp.where(m_acc[...] > -5e29, m_acc[...], 0.0)
    z_scr[:, 0:F] = sv / cnt[:, 0:1]
    z_scr[:, F:2 * F] = mfix
    z_scr[:, 2 * F:3 * F] = sv
    z1 = jnp.maximum(
        jnp.dot(z_scr[...], wc1_ref[...], preferred_element_type=jnp.float32)
        + bc1_ref[...], 0.0)
    z2 = jnp.maximum(
        jnp.dot(z1, wc2_ref[...], preferred_element_type=jnp.float32)
        + bc2_ref[...], 0.0)
    out_ref[...] = (jnp.dot(z2, wc3_ref[...],
                            preferred_element_type=jnp.float32)
                    + bc3_ref[...])


def _pool_mlp(h_list, b2, batch2, wc1, bc1, wc2, bc2, wc3, bc3):
  def body(*refs):
    _pool_body(refs[0:N_SLICES], *refs[N_SLICES:])

  hid2 = wc1.shape[1]
  in_specs = ([pl.BlockSpec((ROW_BLK, 128), lambda i: (i, 0))
               for _ in range(N_SLICES)] + [
      pl.BlockSpec((1, F), lambda i: (0, 0)),
      pl.BlockSpec((ROW_BLK, 1), lambda i: (i, 0)),
      pl.BlockSpec(wc1.shape, lambda i: (0, 0)),
      pl.BlockSpec((1, hid2), lambda i: (0, 0)),
      pl.BlockSpec(wc2.shape, lambda i: (0, 0)),
      pl.BlockSpec((1, HID), lambda i: (0, 0)),
      pl.BlockSpec(wc3.shape, lambda i: (0, 0)),
      pl.BlockSpec((1, OUT_DIM), lambda i: (0, 0)),
  ])
  return pl.pallas_call(
      body,
      grid=(N_ROW_BLKS,),
      in_specs=in_specs,
      out_specs=pl.BlockSpec((NUM_GRAPHS, OUT_DIM), lambda i: (0, 0)),
      out_shape=jax.ShapeDtypeStruct((NUM_GRAPHS, OUT_DIM), jnp.float32),
      scratch_shapes=[
          pltpu.VMEM((NUM_GRAPHS, F), jnp.float32),
          pltpu.VMEM((NUM_GRAPHS, F), jnp.float32),
          pltpu.VMEM((NUM_GRAPHS, 128), jnp.float32),
          pltpu.VMEM((NUM_GRAPHS, 3 * F), jnp.float32),
      ],
  )(*h_list, b2.reshape(1, F), batch2, wc1, bc1.reshape(1, hid2),
    wc2, bc2.reshape(1, HID), wc3, bc3.reshape(1, OUT_DIM))


def _att_matrix(att):
  rows = jnp.arange(F) // HID
  cols = jnp.arange(AC) % HEADS
  att_flat = att.reshape(F)
  return jnp.where(rows[:, None] == cols[None, :], att_flat[:, None], 0.0)


def _gat_layer(x, src, dst, w, a_s, a_d, b_prev, apply_act):
  s_mat = jnp.concatenate([_att_matrix(a_s), _att_matrix(a_d)], axis=1)
  outs = _projection(x, w, s_mat, b_prev.reshape(1, -1), apply_act)
  h_list, as_t, ad_t = outs[:N_SLICES], outs[N_SLICES], outs[N_SLICES + 1]
  ex, den0, den1 = _edge_stats(src, dst, as_t, ad_t)
  alpha = _alpha_kernel(ex, den0, den1, dst)
  return _aggregate(*h_list, src, dst, alpha)


def kernel(x, edge_index, batch, W0, a_s0, a_d0, b0, W1, a_s1, a_d1, b1,
           W2, a_s2, a_d2, b2, Wc1, bc1, Wc2, bc2, Wc3, bc3):
  src, dst = edge_index[0], edge_index[1]
  x_pad = jnp.pad(x, ((0, N_PAD - N), (0, 0)))
  batch2 = jnp.pad(batch, (0, N_PAD - N),
                   constant_values=NUM_GRAPHS).reshape(N_PAD, 1)

  o1 = _gat_layer(x_pad, src, dst, W0, a_s0, a_d0,
                  jnp.zeros((IN_DIM,), jnp.float32), False)
  x1 = jnp.concatenate(o1, axis=1)
  o2 = _gat_layer(x1, src, dst, W1, a_s1, a_d1, b0, True)
  x2 = jnp.concatenate(o2, axis=1)
  o3 = _gat_layer(x2, src, dst, W2, a_s2, a_d2, b1, True)

  return _pool_mlp(o3, b2, batch2, Wc1, bc1, Wc2, bc2, Wc3, bc3)
